# Initial kernel scaffold; baseline (speedup 1.0000x reference)
import jax
import jax.numpy as jnp
from jax.experimental import pallas as pl


def kernel(x, edge_index, W1, b1, W2, b2):
    raise NotImplementedError("write your pallas kernel here")


# speedup vs baseline: 3.0901x; 3.0901x over previous
import functools

import jax
import jax.numpy as jnp
from jax import lax
from jax.experimental import pallas as pl
from jax.experimental.pallas import tpu as pltpu
from jax.experimental.pallas import tpu_sc as plsc

N_NODES = 10000
N_EDGES = 160000
IN_FEATS = 256
HIDDEN = 256
NUM_CLASSES = 64

NC = 2
NS = 16
CHUNK = 80
EPT = N_EDGES // NS
NCHUNK = EPT // CHUNK
ZB = 632
ZLAST = N_NODES - (NS - 1) * ZB


def _linear1(x, w, b):
    blk = 1000
    half = HIDDEN // 2

    def body(x_ref, w_ref, b_ref, lo_ref, hi_ref):
        h = jnp.dot(x_ref[...], w_ref[...], preferred_element_type=jnp.float32)
        h = h + b_ref[...]
        lo_ref[...] = h[:, :half]
        hi_ref[...] = h[:, half:]

    return pl.pallas_call(
        body,
        grid=(N_NODES // blk,),
        in_specs=[
            pl.BlockSpec((blk, IN_FEATS), lambda i: (i, 0)),
            pl.BlockSpec((IN_FEATS, HIDDEN), lambda i: (0, 0)),
            pl.BlockSpec((1, HIDDEN), lambda i: (0, 0)),
        ],
        out_specs=[
            pl.BlockSpec((blk, half), lambda i: (i, 0)),
            pl.BlockSpec((blk, half), lambda i: (i, 0)),
        ],
        out_shape=[jax.ShapeDtypeStruct((N_NODES, half), jnp.float32)] * 2,
    )(x, w, b)


def _linear2(lo, hi, wa, wb, b):
    blk = 1000
    half = NUM_CLASSES // 2

    def body(lo_ref, hi_ref, wa_ref, wb_ref, b_ref, olo_ref, ohi_ref):
        h = jnp.dot(jnp.maximum(lo_ref[...], 0.0), wa_ref[...],
                    preferred_element_type=jnp.float32)
        h = h + jnp.dot(jnp.maximum(hi_ref[...], 0.0), wb_ref[...],
                        preferred_element_type=jnp.float32)
        h = h + b_ref[...]
        olo_ref[...] = h[:, :half]
        ohi_ref[...] = h[:, half:]

    return pl.pallas_call(
        body,
        grid=(N_NODES // blk,),
        in_specs=[
            pl.BlockSpec((blk, HIDDEN // 2), lambda i: (i, 0)),
            pl.BlockSpec((blk, HIDDEN // 2), lambda i: (i, 0)),
            pl.BlockSpec((HIDDEN // 2, NUM_CLASSES), lambda i: (0, 0)),
            pl.BlockSpec((HIDDEN // 2, NUM_CLASSES), lambda i: (0, 0)),
            pl.BlockSpec((1, NUM_CLASSES), lambda i: (0, 0)),
        ],
        out_specs=[
            pl.BlockSpec((blk, half), lambda i: (i, 0)),
            pl.BlockSpec((blk, half), lambda i: (i, 0)),
        ],
        out_shape=[jax.ShapeDtypeStruct((N_NODES, half), jnp.float32)] * 2,
    )(lo, hi, wa, wb, b)


@functools.lru_cache(maxsize=None)
def _make_scatter(d):
    mesh = plsc.VectorSubcoreMesh(core_axis_name="c", subcore_axis_name="s")

    @functools.partial(
        pl.kernel,
        out_type=[jax.ShapeDtypeStruct((N_NODES, d), jnp.float32)] * 2,
        mesh=mesh,
        scratch_types=[
            pltpu.VMEM((CHUNK,), jnp.int32),
            pltpu.VMEM((CHUNK,), jnp.int32),
            pltpu.VMEM((CHUNK, d), jnp.float32),
            pltpu.VMEM_SHARED((N_NODES, d), jnp.float32),
            pltpu.SemaphoreType.DMA,
        ],
        compiler_params=pltpu.CompilerParams(use_tc_tiling_on_sc=False),
    )
    def k(h_lo, h_hi, src_hbm, dst_hbm, z_hbm, out_lo, out_hi,
          sidx, didx, rows, agg, sem):
        c = lax.axis_index("c")
        s = lax.axis_index("s")
        r0 = pl.multiple_of(s * ZB, 8)

        @pl.when(s < NS - 1)
        def _():
            pltpu.sync_copy(z_hbm.at[pl.ds(r0, ZB)], agg.at[pl.ds(r0, ZB)])

        @pl.when(s == NS - 1)
        def _():
            pltpu.sync_copy(z_hbm.at[pl.ds(r0, ZLAST)], agg.at[pl.ds(r0, ZLAST)])

        plsc.subcore_barrier()

        def run_edges(h_hbm):
            def chunk_body(j, carry):
                base = s * EPT + j * CHUNK
                pltpu.sync_copy(src_hbm.at[pl.ds(base, CHUNK)], sidx)
                pltpu.sync_copy(dst_hbm.at[pl.ds(base, CHUNK)], didx)
                pltpu.async_copy(h_hbm.at[sidx], rows, sem).wait()
                pltpu.sync_copy(rows, agg.at[didx], add=True)
                return carry
            lax.fori_loop(0, NCHUNK, chunk_body, 0)

        @pl.when(c == 0)
        def _():
            run_edges(h_lo)

        @pl.when(c == 1)
        def _():
            run_edges(h_hi)

        plsc.subcore_barrier()

        def copy_out(out_hbm):
            @pl.when(s < NS - 1)
            def _():
                pltpu.sync_copy(agg.at[pl.ds(r0, ZB)], out_hbm.at[pl.ds(r0, ZB)])

            @pl.when(s == NS - 1)
            def _():
                pltpu.sync_copy(agg.at[pl.ds(r0, ZLAST)],
                                out_hbm.at[pl.ds(r0, ZLAST)])

        @pl.when(c == 0)
        def _():
            copy_out(out_lo)

        @pl.when(c == 1)
        def _():
            copy_out(out_hi)

    return k


def kernel(x, edge_index, W1, b1, W2, b2):
    ei = edge_index.astype(jnp.int32)
    src = ei[0]
    dst = ei[1]

    lo, hi = _linear1(x, W1, b1.reshape(1, HIDDEN))
    z1 = jnp.zeros((N_NODES, HIDDEN // 2), jnp.float32)
    a_lo, a_hi = _make_scatter(HIDDEN // 2)(lo, hi, src, dst, z1)

    lo2, hi2 = _linear2(a_lo, a_hi, W2[: HIDDEN // 2], W2[HIDDEN // 2:],
                        b2.reshape(1, NUM_CLASSES))
    z2 = jnp.zeros((N_NODES, NUM_CLASSES // 2), jnp.float32)
    o_lo, o_hi = _make_scatter(NUM_CLASSES // 2)(lo2, hi2, src, dst, z2)

    return jnp.concatenate([o_lo, o_hi], axis=1)


# speedup vs baseline: 5.9869x; 1.9374x over previous
import functools

import jax
import jax.numpy as jnp
from jax import lax
from jax.experimental import pallas as pl
from jax.experimental.pallas import tpu as pltpu
from jax.experimental.pallas import tpu_sc as plsc

N_NODES = 10000
N_EDGES = 160000
IN_FEATS = 256
HIDDEN = 256
NUM_CLASSES = 64

NC = 2
NS = 16
CHUNK = 80
EPT = N_EDGES // NS
NCHUNK = EPT // CHUNK
ZB = 632
ZLAST = N_NODES - (NS - 1) * ZB


def _linear1(x, w, b):
    blk = 1000
    half = HIDDEN // 2

    def body(x_ref, w_ref, b_ref, lo_ref, hi_ref):
        h = jnp.dot(x_ref[...], w_ref[...], preferred_element_type=jnp.float32)
        h = h + b_ref[...]
        lo_ref[...] = h[:, :half]
        hi_ref[...] = h[:, half:]

    return pl.pallas_call(
        body,
        grid=(N_NODES // blk,),
        in_specs=[
            pl.BlockSpec((blk, IN_FEATS), lambda i: (i, 0)),
            pl.BlockSpec((IN_FEATS, HIDDEN), lambda i: (0, 0)),
            pl.BlockSpec((1, HIDDEN), lambda i: (0, 0)),
        ],
        out_specs=[
            pl.BlockSpec((blk, half), lambda i: (i, 0)),
            pl.BlockSpec((blk, half), lambda i: (i, 0)),
        ],
        out_shape=[jax.ShapeDtypeStruct((N_NODES, half), jnp.float32)] * 2,
    )(x, w, b)


def _linear2(lo, hi, wa, wb, b):
    blk = 1000
    half = NUM_CLASSES // 2

    def body(lo_ref, hi_ref, wa_ref, wb_ref, b_ref, olo_ref, ohi_ref):
        h = jnp.dot(jnp.maximum(lo_ref[...], 0.0), wa_ref[...],
                    preferred_element_type=jnp.float32)
        h = h + jnp.dot(jnp.maximum(hi_ref[...], 0.0), wb_ref[...],
                        preferred_element_type=jnp.float32)
        h = h + b_ref[...]
        olo_ref[...] = h[:, :half]
        ohi_ref[...] = h[:, half:]

    return pl.pallas_call(
        body,
        grid=(N_NODES // blk,),
        in_specs=[
            pl.BlockSpec((blk, HIDDEN // 2), lambda i: (i, 0)),
            pl.BlockSpec((blk, HIDDEN // 2), lambda i: (i, 0)),
            pl.BlockSpec((HIDDEN // 2, NUM_CLASSES), lambda i: (0, 0)),
            pl.BlockSpec((HIDDEN // 2, NUM_CLASSES), lambda i: (0, 0)),
            pl.BlockSpec((1, NUM_CLASSES), lambda i: (0, 0)),
        ],
        out_specs=[
            pl.BlockSpec((blk, half), lambda i: (i, 0)),
            pl.BlockSpec((blk, half), lambda i: (i, 0)),
        ],
        out_shape=[jax.ShapeDtypeStruct((N_NODES, half), jnp.float32)] * 2,
    )(lo, hi, wa, wb, b)


@functools.lru_cache(maxsize=None)
def _make_scatter(d):
    mesh = plsc.VectorSubcoreMesh(core_axis_name="c", subcore_axis_name="s")

    @functools.partial(
        pl.kernel,
        out_type=[jax.ShapeDtypeStruct((N_NODES, d), jnp.float32)] * 2,
        mesh=mesh,
        scratch_types=[
            pltpu.VMEM((NCHUNK, CHUNK), jnp.int32),
            pltpu.VMEM((NCHUNK, CHUNK), jnp.int32),
            pltpu.VMEM((CHUNK, d), jnp.float32),
            pltpu.VMEM((CHUNK, d), jnp.float32),
            pltpu.VMEM_SHARED((N_NODES, d), jnp.float32),
            pltpu.SemaphoreType.DMA,
            pltpu.SemaphoreType.DMA,
        ],
        compiler_params=pltpu.CompilerParams(use_tc_tiling_on_sc=False),
    )
    def k(h_lo, h_hi, src_hbm, dst_hbm, z_hbm, out_lo, out_hi,
          sidx, didx, rows0, rows1, agg, sem0, sem1):
        c = lax.axis_index("c")
        s = lax.axis_index("s")
        r0 = pl.multiple_of(s * ZB, 8)

        pltpu.sync_copy(src_hbm.at[s], sidx)
        pltpu.sync_copy(dst_hbm.at[s], didx)

        @pl.when(s < NS - 1)
        def _():
            pltpu.sync_copy(z_hbm.at[pl.ds(r0, ZB)], agg.at[pl.ds(r0, ZB)])

        @pl.when(s == NS - 1)
        def _():
            pltpu.sync_copy(z_hbm.at[pl.ds(r0, ZLAST)], agg.at[pl.ds(r0, ZLAST)])

        plsc.subcore_barrier()

        def run_edges(h_hbm):
            def chunk_body(jj, carry):
                j0 = 2 * jj
                j1 = 2 * jj + 1
                g0 = pltpu.async_copy(h_hbm.at[sidx.at[j0]], rows0, sem0)
                g1 = pltpu.async_copy(h_hbm.at[sidx.at[j1]], rows1, sem1)
                g0.wait()
                pltpu.sync_copy(rows0, agg.at[didx.at[j0]], add=True)
                g1.wait()
                pltpu.sync_copy(rows1, agg.at[didx.at[j1]], add=True)
                return carry
            lax.fori_loop(0, NCHUNK // 2, chunk_body, 0)
            g0 = pltpu.async_copy(h_hbm.at[sidx.at[NCHUNK - 1]], rows0, sem0)
            g0.wait()
            pltpu.sync_copy(rows0, agg.at[didx.at[NCHUNK - 1]], add=True)

        @pl.when(c == 0)
        def _():
            run_edges(h_lo)

        @pl.when(c == 1)
        def _():
            run_edges(h_hi)

        plsc.subcore_barrier()

        def copy_out(out_hbm):
            @pl.when(s < NS - 1)
            def _():
                pltpu.sync_copy(agg.at[pl.ds(r0, ZB)], out_hbm.at[pl.ds(r0, ZB)])

            @pl.when(s == NS - 1)
            def _():
                pltpu.sync_copy(agg.at[pl.ds(r0, ZLAST)],
                                out_hbm.at[pl.ds(r0, ZLAST)])

        @pl.when(c == 0)
        def _():
            copy_out(out_lo)

        @pl.when(c == 1)
        def _():
            copy_out(out_hi)

    return k


def kernel(x, edge_index, W1, b1, W2, b2):
    ei = edge_index.astype(jnp.int32)
    src = ei[0].reshape(NS, NCHUNK, CHUNK)
    dst = ei[1].reshape(NS, NCHUNK, CHUNK)

    lo, hi = _linear1(x, W1, b1.reshape(1, HIDDEN))
    z1 = jnp.zeros((N_NODES, HIDDEN // 2), jnp.float32)
    a_lo, a_hi = _make_scatter(HIDDEN // 2)(lo, hi, src, dst, z1)

    lo2, hi2 = _linear2(a_lo, a_hi, W2[: HIDDEN // 2], W2[HIDDEN // 2:],
                        b2.reshape(1, NUM_CLASSES))
    z2 = jnp.zeros((N_NODES, NUM_CLASSES // 2), jnp.float32)
    o_lo, o_hi = _make_scatter(NUM_CLASSES // 2)(lo2, hi2, src, dst, z2)

    return jnp.concatenate([o_lo, o_hi], axis=1)


# speedup vs baseline: 6.3974x; 1.0686x over previous
import functools

import jax
import jax.numpy as jnp
from jax import lax
from jax.experimental import pallas as pl
from jax.experimental.pallas import tpu as pltpu
from jax.experimental.pallas import tpu_sc as plsc

N_NODES = 10000
N_EDGES = 160000
IN_FEATS = 256
HIDDEN = 256
NUM_CLASSES = 64

NC = 2
NS = 16
CHUNK = 80
NBUF = 3
EPT = N_EDGES // NS
NCHUNK = EPT // CHUNK
ZB = 632
ZLAST = N_NODES - (NS - 1) * ZB


def _linear1(x, w, b):
    blk = 1000
    half = HIDDEN // 2

    def body(x_ref, w_ref, b_ref, lo_ref, hi_ref):
        h = jnp.dot(x_ref[...], w_ref[...], preferred_element_type=jnp.float32)
        h = h + b_ref[...]
        lo_ref[...] = h[:, :half]
        hi_ref[...] = h[:, half:]

    return pl.pallas_call(
        body,
        grid=(N_NODES // blk,),
        in_specs=[
            pl.BlockSpec((blk, IN_FEATS), lambda i: (i, 0)),
            pl.BlockSpec((IN_FEATS, HIDDEN), lambda i: (0, 0)),
            pl.BlockSpec((1, HIDDEN), lambda i: (0, 0)),
        ],
        out_specs=[
            pl.BlockSpec((blk, half), lambda i: (i, 0)),
            pl.BlockSpec((blk, half), lambda i: (i, 0)),
        ],
        out_shape=[jax.ShapeDtypeStruct((N_NODES, half), jnp.float32)] * 2,
    )(x, w, b)


def _linear2(lo, hi, wa, wb, b):
    blk = 1000
    half = NUM_CLASSES // 2

    def body(lo_ref, hi_ref, wa_ref, wb_ref, b_ref, olo_ref, ohi_ref):
        h = jnp.dot(jnp.maximum(lo_ref[...], 0.0), wa_ref[...],
                    preferred_element_type=jnp.float32)
        h = h + jnp.dot(jnp.maximum(hi_ref[...], 0.0), wb_ref[...],
                        preferred_element_type=jnp.float32)
        h = h + b_ref[...]
        olo_ref[...] = h[:, :half]
        ohi_ref[...] = h[:, half:]

    return pl.pallas_call(
        body,
        grid=(N_NODES // blk,),
        in_specs=[
            pl.BlockSpec((blk, HIDDEN // 2), lambda i: (i, 0)),
            pl.BlockSpec((blk, HIDDEN // 2), lambda i: (i, 0)),
            pl.BlockSpec((HIDDEN // 2, NUM_CLASSES), lambda i: (0, 0)),
            pl.BlockSpec((HIDDEN // 2, NUM_CLASSES), lambda i: (0, 0)),
            pl.BlockSpec((1, NUM_CLASSES), lambda i: (0, 0)),
        ],
        out_specs=[
            pl.BlockSpec((blk, half), lambda i: (i, 0)),
            pl.BlockSpec((blk, half), lambda i: (i, 0)),
        ],
        out_shape=[jax.ShapeDtypeStruct((N_NODES, half), jnp.float32)] * 2,
    )(lo, hi, wa, wb, b)


@functools.lru_cache(maxsize=None)
def _make_scatter(d):
    mesh = plsc.VectorSubcoreMesh(core_axis_name="c", subcore_axis_name="s")

    @functools.partial(
        pl.kernel,
        out_type=[jax.ShapeDtypeStruct((N_NODES, d), jnp.float32)] * 2,
        mesh=mesh,
        scratch_types=[
            pltpu.VMEM((NCHUNK, CHUNK), jnp.int32),
            pltpu.VMEM((NCHUNK, CHUNK), jnp.int32),
            [pltpu.VMEM((CHUNK, d), jnp.float32)] * NBUF,
            pltpu.VMEM_SHARED((N_NODES, d), jnp.float32),
            [pltpu.SemaphoreType.DMA] * NBUF,
        ],
        compiler_params=pltpu.CompilerParams(use_tc_tiling_on_sc=False),
    )
    def k(h_lo, h_hi, src_hbm, dst_hbm, z_hbm, out_lo, out_hi,
          sidx, didx, rows, agg, sems):
        c = lax.axis_index("c")
        s = lax.axis_index("s")
        r0 = pl.multiple_of(s * ZB, 8)

        pltpu.sync_copy(src_hbm.at[s], sidx)
        pltpu.sync_copy(dst_hbm.at[s], didx)

        @pl.when(s < NS - 1)
        def _():
            pltpu.sync_copy(z_hbm.at[pl.ds(r0, ZB)], agg.at[pl.ds(r0, ZB)])

        @pl.when(s == NS - 1)
        def _():
            pltpu.sync_copy(z_hbm.at[pl.ds(r0, ZLAST)], agg.at[pl.ds(r0, ZLAST)])

        plsc.subcore_barrier()

        def run_edges(h_hbm):
            def chunk_body(jj, carry):
                base = NBUF * jj
                gs = [pltpu.async_copy(h_hbm.at[sidx.at[base + b]],
                                       rows[b], sems[b])
                      for b in range(NBUF)]
                for b in range(NBUF):
                    gs[b].wait()
                    pltpu.sync_copy(rows[b], agg.at[didx.at[base + b]],
                                    add=True)
                return carry
            lax.fori_loop(0, NCHUNK // NBUF, chunk_body, 0)
            for j in range(NCHUNK - NCHUNK % NBUF, NCHUNK):
                g = pltpu.async_copy(h_hbm.at[sidx.at[j]], rows[0], sems[0])
                g.wait()
                pltpu.sync_copy(rows[0], agg.at[didx.at[j]], add=True)

        @pl.when(c == 0)
        def _():
            run_edges(h_lo)

        @pl.when(c == 1)
        def _():
            run_edges(h_hi)

        plsc.subcore_barrier()

        def copy_out(out_hbm):
            @pl.when(s < NS - 1)
            def _():
                pltpu.sync_copy(agg.at[pl.ds(r0, ZB)], out_hbm.at[pl.ds(r0, ZB)])

            @pl.when(s == NS - 1)
            def _():
                pltpu.sync_copy(agg.at[pl.ds(r0, ZLAST)],
                                out_hbm.at[pl.ds(r0, ZLAST)])

        @pl.when(c == 0)
        def _():
            copy_out(out_lo)

        @pl.when(c == 1)
        def _():
            copy_out(out_hi)

    return k


def kernel(x, edge_index, W1, b1, W2, b2):
    ei = edge_index.astype(jnp.int32)
    src = ei[0].reshape(NS, NCHUNK, CHUNK)
    dst = ei[1].reshape(NS, NCHUNK, CHUNK)

    lo, hi = _linear1(x, W1, b1.reshape(1, HIDDEN))
    z1 = jnp.zeros((N_NODES, HIDDEN // 2), jnp.float32)
    a_lo, a_hi = _make_scatter(HIDDEN // 2)(lo, hi, src, dst, z1)

    lo2, hi2 = _linear2(a_lo, a_hi, W2[: HIDDEN // 2], W2[HIDDEN // 2:],
                        b2.reshape(1, NUM_CLASSES))
    z2 = jnp.zeros((N_NODES, NUM_CLASSES // 2), jnp.float32)
    o_lo, o_hi = _make_scatter(NUM_CLASSES // 2)(lo2, hi2, src, dst, z2)

    return jnp.concatenate([o_lo, o_hi], axis=1)
